# baseline (speedup 1.0000x reference)
import jax
import jax.numpy as jnp
from jax.experimental import pallas as pl
from jax.experimental.pallas import tpu as pltpu

_SMEM = getattr(pltpu, "SMEM", None)
if _SMEM is None:
    _SMEM = pltpu.TPUMemorySpace.SMEM

STRIPE = 512
KCHUNK = 4


def _scatter_body(W, H, M, inx_ref, fx_ref, fy_ref, m_ref, acc_ref):
    k = pl.program_id(1)
    s = pl.program_id(2)

    @pl.when(s == 0)
    def _zero():
        acc_ref[...] = jnp.zeros_like(acc_ref)

    base = s * STRIPE
    k_lo = k * M
    Wf = jnp.float32(W)
    Hf = jnp.float32(H)

    def pix(i, carry):
        p = base + i
        y = p // W
        x = p - y * W
        sub = i // 64
        lane = i - sub * 64
        fx = fx_ref[0, 0, sub, lane] + x.astype(jnp.float32)
        fy = fy_ref[0, 0, sub, lane] + y.astype(jnp.float32)
        cx0 = jnp.floor(fx)
        cy0 = jnp.floor(fy)

        corners = []
        for dx in (0.0, 1.0):
            for dy in (0.0, 1.0):
                pxf = cx0 + dx
                pyf = cy0 + dy
                vx = (pxf >= 0.0) & (pxf < Wf)
                vy = (pyf >= 0.0) & (pyf < Hf)
                corners.append((pxf, pyf, vx & vy))

        any_v = corners[0][2] | corners[1][2] | corners[2][2] | corners[3][2]

        @pl.when(any_v)
        def _do():
            wexp = jnp.exp(m_ref[0, 0, i // 64, i % 64])
            row32 = inx_ref[0, pl.ds(i, 1), :] * wexp
            row33 = jnp.concatenate(
                [row32, jnp.full((1, 1), wexp, jnp.float32)], axis=1
            )
            for pxf, pyf, v in corners:
                ix = pxf.astype(jnp.int32)
                iy = pyf.astype(jnp.int32)
                idx = iy * W + ix
                in_chunk = v & (idx >= k_lo) & (idx < k_lo + M)

                @pl.when(in_chunk)
                def _rmw():
                    wgt = (1.0 - jnp.abs(fx - pxf)) * (1.0 - jnp.abs(fy - pyf))
                    loc = idx - k_lo
                    prev = acc_ref[0, pl.ds(loc, 1), :]
                    acc_ref[0, pl.ds(loc, 1), :] = prev + wgt * row33

        return carry

    jax.lax.fori_loop(0, STRIPE, pix, 0)


def _normalize_body(acc_ref, out_ref):
    a = acc_ref[...]
    out_ref[...] = a[:, :, :-1] / (a[:, :, -1:] + 1e-07)


def kernel(tenIn, tenFlow, tenMetric):
    N, C, H, W = tenIn.shape
    HW = H * W
    M = HW // KCHUNK
    NS = HW // STRIPE

    in_nhwc = jnp.transpose(tenIn, (0, 2, 3, 1)).reshape(N, HW, C)
    flowx = tenFlow[:, 0, :, :].reshape(N, NS, 8, 64)
    flowy = tenFlow[:, 1, :, :].reshape(N, NS, 8, 64)
    metric = tenMetric.reshape(N, NS, 8, 64)

    acc = pl.pallas_call(
        lambda *refs: _scatter_body(W, H, M, *refs),
        grid=(N, KCHUNK, NS),
        in_specs=[
            pl.BlockSpec((1, STRIPE, C), lambda n, k, s: (n, s, 0)),
            pl.BlockSpec((1, 1, 8, 64), lambda n, k, s: (n, s, 0, 0), memory_space=_SMEM),
            pl.BlockSpec((1, 1, 8, 64), lambda n, k, s: (n, s, 0, 0), memory_space=_SMEM),
            pl.BlockSpec((1, 1, 8, 64), lambda n, k, s: (n, s, 0, 0), memory_space=_SMEM),
        ],
        out_specs=pl.BlockSpec((1, M, C + 1), lambda n, k, s: (n, k, 0)),
        out_shape=jax.ShapeDtypeStruct((N, HW, C + 1), jnp.float32),
    )(in_nhwc, flowx, flowy, metric)

    out = pl.pallas_call(
        _normalize_body,
        grid=(N, 16),
        in_specs=[pl.BlockSpec((1, HW // 16, C + 1), lambda n, j: (n, j, 0))],
        out_specs=pl.BlockSpec((1, HW // 16, C), lambda n, j: (n, j, 0)),
        out_shape=jax.ShapeDtypeStruct((N, HW, C), jnp.float32),
    )(acc)

    return jnp.transpose(out.reshape(N, H, W, C), (0, 3, 1, 2))
